# baseline (speedup 1.0000x reference)
import functools

import jax
import jax.numpy as jnp
from jax import lax
from jax.experimental import pallas as pl
from jax.experimental.pallas import tpu as pltpu
from jax.experimental.pallas import tpu_sc as plsc

N = 10000
E = 320000
LANES = 128
ROWS = E // LANES
NC = 2
NS = 16
NW = NC * NS
RPW = ROWS // NW
EXTRA = ROWS - RPW * NW
NPAD = 10240
SEG = NPAD // NS
KIF = 16


def _sc_degrees_body(e2, degs, srcb, dstb, oneb, zb, deg_o, deg_i, sem):
  c = lax.axis_index("c")
  s = lax.axis_index("s")
  w = s * NC + c
  one16 = jnp.full((16,), 1.0, jnp.float32)
  zero16 = jnp.zeros((16,), jnp.float32)
  for k in range(LANES // 16):
    oneb[pl.ds(k * 16, 16)] = one16
  for k in range(SEG // 16):
    zb[pl.ds(k * 16, 16)] = zero16
  pltpu.sync_copy(zb, deg_o.at[pl.ds(s * SEG, SEG)])
  pltpu.sync_copy(zb, deg_i.at[pl.ds(s * SEG, SEG)])
  base = w * RPW
  pltpu.sync_copy(e2.at[0, pl.ds(base, RPW)], srcb.at[pl.ds(0, RPW)])
  pltpu.sync_copy(e2.at[1, pl.ds(base, RPW)], dstb.at[pl.ds(0, RPW)])

  @pl.when(w < EXTRA)
  def _():
    pltpu.sync_copy(e2.at[0, NW * RPW + w], srcb.at[RPW])
    pltpu.sync_copy(e2.at[1, NW * RPW + w], dstb.at[RPW])

  plsc.subcore_barrier()

  def body(j, carry):
    pltpu.async_copy(oneb, deg_o.at[srcb.at[j]], sem, add=True)
    pltpu.async_copy(oneb, deg_i.at[dstb.at[j]], sem, add=True)

    @pl.when(j >= KIF)
    def _():
      pltpu.make_async_copy(oneb, deg_o.at[srcb.at[j - KIF]], sem).wait()
      pltpu.make_async_copy(oneb, deg_i.at[dstb.at[j - KIF]], sem).wait()

    return carry

  lax.fori_loop(0, RPW, body, 0)

  @pl.when(w < EXTRA)
  def _():
    pltpu.async_copy(oneb, deg_o.at[srcb.at[RPW]], sem, add=True)
    pltpu.async_copy(oneb, deg_i.at[dstb.at[RPW]], sem, add=True)

  for t in range(KIF):
    jj = RPW - KIF + t
    pltpu.make_async_copy(oneb, deg_o.at[srcb.at[jj]], sem).wait()
    pltpu.make_async_copy(oneb, deg_i.at[dstb.at[jj]], sem).wait()

  @pl.when(w < EXTRA)
  def _():
    pltpu.make_async_copy(oneb, deg_o.at[srcb.at[RPW]], sem).wait()
    pltpu.make_async_copy(oneb, deg_i.at[dstb.at[RPW]], sem).wait()

  plsc.subcore_barrier()
  pltpu.sync_copy(deg_o.at[pl.ds(s * SEG, SEG)], degs.at[c, 0, pl.ds(s * SEG, SEG)])
  pltpu.sync_copy(deg_i.at[pl.ds(s * SEG, SEG)], degs.at[c, 1, pl.ds(s * SEG, SEG)])


def _newton_rsqrt(d):
  dm = jnp.maximum(d, 1.0)
  i = plsc.bitcast(dm, jnp.int32)
  i = jnp.full((16,), 0x5F3759DF, jnp.int32) - lax.shift_right_logical(i, 1)
  y = plsc.bitcast(i, jnp.float32)
  half = dm * 0.5
  for _ in range(3):
    y = y * (1.5 - half * y * y)
  return jnp.where(d > 0, y, 0.0)


def _sc_scatter_body(e2, u2, degs, accp, srcb, dstb, msga, msgb, ga_loc, gb_loc,
                     useg_a, useg_b, d0a, d0b, gseg_a, gseg_b, zb, ga_sh, gb_sh,
                     acc_a, acc_b, sem):
  c = lax.axis_index("c")
  s = lax.axis_index("s")
  w = s * NC + c
  t0 = s * SEG
  zero16 = jnp.zeros((16,), jnp.float32)
  for k in range(SEG // 16):
    zb[pl.ds(k * 16, 16)] = zero16
  pltpu.sync_copy(zb, acc_a.at[pl.ds(t0, SEG)])
  pltpu.sync_copy(zb, acc_b.at[pl.ds(t0, SEG)])
  pltpu.sync_copy(u2.at[0, pl.ds(t0, SEG)], useg_a)
  pltpu.sync_copy(u2.at[1, pl.ds(t0, SEG)], useg_b)
  pltpu.sync_copy(degs.at[0, 0, pl.ds(t0, SEG)], d0a)
  pltpu.sync_copy(degs.at[1, 0, pl.ds(t0, SEG)], d0b)
  base = w * RPW
  pltpu.sync_copy(e2.at[0, pl.ds(base, RPW)], srcb.at[pl.ds(0, RPW)])
  pltpu.sync_copy(e2.at[1, pl.ds(base, RPW)], dstb.at[pl.ds(0, RPW)])

  @pl.when(w < EXTRA)
  def _():
    pltpu.sync_copy(e2.at[0, NW * RPW + w], srcb.at[RPW])
    pltpu.sync_copy(e2.at[1, NW * RPW + w], dstb.at[RPW])

  for k in range(SEG // 16):
    sl = pl.ds(k * 16, 16)
    ns = _newton_rsqrt(d0a[sl] + d0b[sl])
    gseg_a[sl] = useg_a[sl] * ns
    gseg_b[sl] = useg_b[sl] * ns
  pltpu.sync_copy(gseg_a, ga_sh.at[pl.ds(t0, SEG)])
  pltpu.sync_copy(gseg_b, gb_sh.at[pl.ds(t0, SEG)])
  plsc.subcore_barrier()
  pltpu.sync_copy(ga_sh, ga_loc)
  pltpu.sync_copy(gb_sh, gb_loc)

  def gather_row(j):
    for k in range(LANES // 16):
      idx16 = srcb[j, pl.ds(k * 16, 16)]
      msga[j, pl.ds(k * 16, 16)] = plsc.load_gather(ga_loc, [idx16])
      msgb[j, pl.ds(k * 16, 16)] = plsc.load_gather(gb_loc, [idx16])

  def sbody(j, carry):
    gather_row(j)
    pltpu.async_copy(msga.at[j], acc_a.at[dstb.at[j]], sem, add=True)
    pltpu.async_copy(msgb.at[j], acc_b.at[dstb.at[j]], sem, add=True)

    @pl.when(j >= KIF)
    def _():
      jj = j - KIF
      pltpu.make_async_copy(msga.at[jj], acc_a.at[dstb.at[jj]], sem).wait()
      pltpu.make_async_copy(msgb.at[jj], acc_b.at[dstb.at[jj]], sem).wait()

    return carry

  lax.fori_loop(0, RPW, sbody, 0)

  @pl.when(w < EXTRA)
  def _():
    gather_row(RPW)
    pltpu.async_copy(msga.at[RPW], acc_a.at[dstb.at[RPW]], sem, add=True)
    pltpu.async_copy(msgb.at[RPW], acc_b.at[dstb.at[RPW]], sem, add=True)

  for t in range(KIF):
    jj = RPW - KIF + t
    pltpu.make_async_copy(msga.at[jj], acc_a.at[dstb.at[jj]], sem).wait()
    pltpu.make_async_copy(msgb.at[jj], acc_b.at[dstb.at[jj]], sem).wait()

  @pl.when(w < EXTRA)
  def _():
    pltpu.make_async_copy(msga.at[RPW], acc_a.at[dstb.at[RPW]], sem).wait()
    pltpu.make_async_copy(msgb.at[RPW], acc_b.at[dstb.at[RPW]], sem).wait()

  plsc.subcore_barrier()
  pltpu.sync_copy(acc_a.at[pl.ds(t0, SEG)], accp.at[c, 0, pl.ds(t0, SEG)])
  pltpu.sync_copy(acc_b.at[pl.ds(t0, SEG)], accp.at[c, 1, pl.ds(t0, SEG)])


_sc_degrees = functools.partial(
    pl.kernel,
    out_type=jax.ShapeDtypeStruct((NC, 2, NPAD), jnp.float32),
    mesh=plsc.VectorSubcoreMesh(
        core_axis_name="c", subcore_axis_name="s", num_cores=NC, num_subcores=NS
    ),
    compiler_params=pltpu.CompilerParams(use_tc_tiling_on_sc=False, needs_layout_passes=False),
    scratch_types=[
        pltpu.VMEM((RPW + 1, LANES), jnp.int32),
        pltpu.VMEM((RPW + 1, LANES), jnp.int32),
        pltpu.VMEM((LANES,), jnp.float32),
        pltpu.VMEM((SEG,), jnp.float32),
        pltpu.VMEM_SHARED((NPAD,), jnp.float32),
        pltpu.VMEM_SHARED((NPAD,), jnp.float32),
        pltpu.SemaphoreType.DMA,
    ],
)(_sc_degrees_body)


_sc_scatter = functools.partial(
    pl.kernel,
    out_type=jax.ShapeDtypeStruct((NC, 2, NPAD), jnp.float32),
    mesh=plsc.VectorSubcoreMesh(
        core_axis_name="c", subcore_axis_name="s", num_cores=NC, num_subcores=NS
    ),
    compiler_params=pltpu.CompilerParams(use_tc_tiling_on_sc=False, needs_layout_passes=False),
    scratch_types=[
        pltpu.VMEM((RPW + 1, LANES), jnp.int32),
        pltpu.VMEM((RPW + 1, LANES), jnp.int32),
        pltpu.VMEM((RPW + 1, LANES), jnp.float32),
        pltpu.VMEM((RPW + 1, LANES), jnp.float32),
        pltpu.VMEM((NPAD,), jnp.float32),
        pltpu.VMEM((NPAD,), jnp.float32),
        pltpu.VMEM((SEG,), jnp.float32),
        pltpu.VMEM((SEG,), jnp.float32),
        pltpu.VMEM((SEG,), jnp.float32),
        pltpu.VMEM((SEG,), jnp.float32),
        pltpu.VMEM((SEG,), jnp.float32),
        pltpu.VMEM((SEG,), jnp.float32),
        pltpu.VMEM((SEG,), jnp.float32),
        pltpu.VMEM_SHARED((NPAD,), jnp.float32),
        pltpu.VMEM_SHARED((NPAD,), jnp.float32),
        pltpu.VMEM_SHARED((NPAD,), jnp.float32),
        pltpu.VMEM_SHARED((NPAD,), jnp.float32),
        pltpu.SemaphoreType.DMA,
    ],
)(_sc_scatter_body)


def _tc_u_body(x_ref, we_ref, fw_ref, wg_ref, cw_ref, u2_ref):
  v = jnp.concatenate(
      [
          jnp.dot(we_ref[...], fw_ref[...], preferred_element_type=jnp.float32),
          jnp.dot(wg_ref[...], cw_ref[...], preferred_element_type=jnp.float32),
      ],
      axis=1,
  )
  u_t = lax.dot_general(v, x_ref[...], (((0,), (1,)), ((), ())),
                        preferred_element_type=jnp.float32)
  u2_ref[...] = jnp.concatenate(
      [u_t, jnp.zeros((2, NPAD - N), jnp.float32)], axis=1
  )


def _tc_final_body(accp_ref, degs_ref, be_ref, fw_ref, fb_ref, bg_ref, cw_ref,
                   cb_ref, y_ref, s_ref):
  deg_i = degs_ref[0, 1, :N] + degs_ref[1, 1, :N]
  nd = jnp.where(deg_i > 0, lax.rsqrt(jnp.maximum(deg_i, 1.0)), 0.0)
  acc_a = accp_ref[0, 0, :N] + accp_ref[1, 0, :N]
  acc_b = accp_ref[0, 1, :N] + accp_ref[1, 1, :N]
  c_est = jnp.dot(be_ref[...][None, :], fw_ref[...],
                  preferred_element_type=jnp.float32)[0, 0] + fb_ref[0]
  c_gnn = jnp.dot(bg_ref[...][None, :], cw_ref[...],
                  preferred_element_type=jnp.float32)[0, 0] + cb_ref[0]
  s_ref[...] = (acc_a * nd + c_est)[:, None]
  y_ref[...] = (acc_b * nd + c_gnn)[:, None]


_tc_u = pl.pallas_call(
    _tc_u_body,
    out_shape=jax.ShapeDtypeStruct((2, NPAD), jnp.float32),
)

_tc_final = pl.pallas_call(
    _tc_final_body,
    out_shape=[
        jax.ShapeDtypeStruct((N, 1), jnp.float32),
        jax.ShapeDtypeStruct((N, 1), jnp.float32),
    ],
)


def kernel(x, edge_index, W_est, b_est, fc_w, fc_b, W_gnn, b_gnn, cls_w, cls_b):
  e2 = edge_index.reshape(2, ROWS, LANES)
  u2 = _tc_u(x, W_est, fc_w, W_gnn, cls_w)
  degs = _sc_degrees(e2)
  accp = _sc_scatter(e2, u2, degs)
  y, s = _tc_final(accp, degs, b_est, fc_w, fc_b, b_gnn, cls_w, cls_b)
  return (y, s)


# speedup vs baseline: 1.0040x; 1.0040x over previous
import functools

import jax
import jax.numpy as jnp
from jax import lax
from jax.experimental import pallas as pl
from jax.experimental.pallas import tpu as pltpu
from jax.experimental.pallas import tpu_sc as plsc

N = 10000
E = 320000
LANES = 128
ROWS = E // LANES
NC = 2
NS = 16
NW = NC * NS
RPW = ROWS // NW
EXTRA = ROWS - RPW * NW
NPAD = 10240
SEG = NPAD // NS
KIF = 8


def _sc_degrees_body(e2, degs, srcb, dstb, oneb, zb, deg_o, deg_i, sem):
  c = lax.axis_index("c")
  s = lax.axis_index("s")
  w = s * NC + c
  one16 = jnp.full((16,), 1.0, jnp.float32)
  zero16 = jnp.zeros((16,), jnp.float32)
  for k in range(LANES // 16):
    oneb[pl.ds(k * 16, 16)] = one16
  for k in range(SEG // 16):
    zb[pl.ds(k * 16, 16)] = zero16
  pltpu.sync_copy(zb, deg_o.at[pl.ds(s * SEG, SEG)])
  pltpu.sync_copy(zb, deg_i.at[pl.ds(s * SEG, SEG)])
  base = w * RPW
  pltpu.sync_copy(e2.at[0, pl.ds(base, RPW)], srcb.at[pl.ds(0, RPW)])
  pltpu.sync_copy(e2.at[1, pl.ds(base, RPW)], dstb.at[pl.ds(0, RPW)])

  @pl.when(w < EXTRA)
  def _():
    pltpu.sync_copy(e2.at[0, NW * RPW + w], srcb.at[RPW])
    pltpu.sync_copy(e2.at[1, NW * RPW + w], dstb.at[RPW])

  plsc.subcore_barrier()

  def body(j, carry):
    pltpu.async_copy(oneb, deg_o.at[srcb.at[j]], sem, add=True)
    pltpu.async_copy(oneb, deg_i.at[dstb.at[j]], sem, add=True)

    @pl.when(j >= KIF)
    def _():
      pltpu.make_async_copy(oneb, deg_o.at[srcb.at[j - KIF]], sem).wait()
      pltpu.make_async_copy(oneb, deg_i.at[dstb.at[j - KIF]], sem).wait()

    return carry

  lax.fori_loop(0, RPW, body, 0)

  @pl.when(w < EXTRA)
  def _():
    pltpu.async_copy(oneb, deg_o.at[srcb.at[RPW]], sem, add=True)
    pltpu.async_copy(oneb, deg_i.at[dstb.at[RPW]], sem, add=True)

  for t in range(KIF):
    jj = RPW - KIF + t
    pltpu.make_async_copy(oneb, deg_o.at[srcb.at[jj]], sem).wait()
    pltpu.make_async_copy(oneb, deg_i.at[dstb.at[jj]], sem).wait()

  @pl.when(w < EXTRA)
  def _():
    pltpu.make_async_copy(oneb, deg_o.at[srcb.at[RPW]], sem).wait()
    pltpu.make_async_copy(oneb, deg_i.at[dstb.at[RPW]], sem).wait()

  plsc.subcore_barrier()
  pltpu.sync_copy(deg_o.at[pl.ds(s * SEG, SEG)], degs.at[c, 0, pl.ds(s * SEG, SEG)])
  pltpu.sync_copy(deg_i.at[pl.ds(s * SEG, SEG)], degs.at[c, 1, pl.ds(s * SEG, SEG)])


def _newton_rsqrt(d):
  dm = jnp.maximum(d, 1.0)
  i = plsc.bitcast(dm, jnp.int32)
  i = jnp.full((16,), 0x5F3759DF, jnp.int32) - lax.shift_right_logical(i, 1)
  y = plsc.bitcast(i, jnp.float32)
  half = dm * 0.5
  for _ in range(3):
    y = y * (1.5 - half * y * y)
  return jnp.where(d > 0, y, 0.0)


def _sc_scatter_body(e2, u2, degs, accp, srcb, dstb, msga, msgb, ga_loc, gb_loc,
                     useg_a, useg_b, d0a, d0b, gseg_a, gseg_b, zb, ga_sh, gb_sh,
                     acc_a, acc_b, sem):
  c = lax.axis_index("c")
  s = lax.axis_index("s")
  w = s * NC + c
  t0 = s * SEG
  zero16 = jnp.zeros((16,), jnp.float32)
  for k in range(SEG // 16):
    zb[pl.ds(k * 16, 16)] = zero16
  pltpu.sync_copy(zb, acc_a.at[pl.ds(t0, SEG)])
  pltpu.sync_copy(zb, acc_b.at[pl.ds(t0, SEG)])
  pltpu.sync_copy(u2.at[0, pl.ds(t0, SEG)], useg_a)
  pltpu.sync_copy(u2.at[1, pl.ds(t0, SEG)], useg_b)
  pltpu.sync_copy(degs.at[0, 0, pl.ds(t0, SEG)], d0a)
  pltpu.sync_copy(degs.at[1, 0, pl.ds(t0, SEG)], d0b)
  base = w * RPW
  pltpu.sync_copy(e2.at[0, pl.ds(base, RPW)], srcb.at[pl.ds(0, RPW)])
  pltpu.sync_copy(e2.at[1, pl.ds(base, RPW)], dstb.at[pl.ds(0, RPW)])

  @pl.when(w < EXTRA)
  def _():
    pltpu.sync_copy(e2.at[0, NW * RPW + w], srcb.at[RPW])
    pltpu.sync_copy(e2.at[1, NW * RPW + w], dstb.at[RPW])

  for k in range(SEG // 16):
    sl = pl.ds(k * 16, 16)
    ns = _newton_rsqrt(d0a[sl] + d0b[sl])
    gseg_a[sl] = useg_a[sl] * ns
    gseg_b[sl] = useg_b[sl] * ns
  pltpu.sync_copy(gseg_a, ga_sh.at[pl.ds(t0, SEG)])
  pltpu.sync_copy(gseg_b, gb_sh.at[pl.ds(t0, SEG)])
  plsc.subcore_barrier()
  pltpu.sync_copy(ga_sh, ga_loc)
  pltpu.sync_copy(gb_sh, gb_loc)

  def gather_row(j):
    for k in range(LANES // 16):
      idx16 = srcb[j, pl.ds(k * 16, 16)]
      msga[j, pl.ds(k * 16, 16)] = plsc.load_gather(ga_loc, [idx16])
      msgb[j, pl.ds(k * 16, 16)] = plsc.load_gather(gb_loc, [idx16])

  def sbody(j, carry):
    gather_row(j)
    pltpu.async_copy(msga.at[j], acc_a.at[dstb.at[j]], sem, add=True)
    pltpu.async_copy(msgb.at[j], acc_b.at[dstb.at[j]], sem, add=True)

    @pl.when(j >= KIF)
    def _():
      jj = j - KIF
      pltpu.make_async_copy(msga.at[jj], acc_a.at[dstb.at[jj]], sem).wait()
      pltpu.make_async_copy(msgb.at[jj], acc_b.at[dstb.at[jj]], sem).wait()

    return carry

  lax.fori_loop(0, RPW, sbody, 0)

  @pl.when(w < EXTRA)
  def _():
    gather_row(RPW)
    pltpu.async_copy(msga.at[RPW], acc_a.at[dstb.at[RPW]], sem, add=True)
    pltpu.async_copy(msgb.at[RPW], acc_b.at[dstb.at[RPW]], sem, add=True)

  for t in range(KIF):
    jj = RPW - KIF + t
    pltpu.make_async_copy(msga.at[jj], acc_a.at[dstb.at[jj]], sem).wait()
    pltpu.make_async_copy(msgb.at[jj], acc_b.at[dstb.at[jj]], sem).wait()

  @pl.when(w < EXTRA)
  def _():
    pltpu.make_async_copy(msga.at[RPW], acc_a.at[dstb.at[RPW]], sem).wait()
    pltpu.make_async_copy(msgb.at[RPW], acc_b.at[dstb.at[RPW]], sem).wait()

  plsc.subcore_barrier()
  pltpu.sync_copy(acc_a.at[pl.ds(t0, SEG)], accp.at[c, 0, pl.ds(t0, SEG)])
  pltpu.sync_copy(acc_b.at[pl.ds(t0, SEG)], accp.at[c, 1, pl.ds(t0, SEG)])


_sc_degrees = functools.partial(
    pl.kernel,
    out_type=jax.ShapeDtypeStruct((NC, 2, NPAD), jnp.float32),
    mesh=plsc.VectorSubcoreMesh(
        core_axis_name="c", subcore_axis_name="s", num_cores=NC, num_subcores=NS
    ),
    compiler_params=pltpu.CompilerParams(use_tc_tiling_on_sc=False, needs_layout_passes=False),
    scratch_types=[
        pltpu.VMEM((RPW + 1, LANES), jnp.int32),
        pltpu.VMEM((RPW + 1, LANES), jnp.int32),
        pltpu.VMEM((LANES,), jnp.float32),
        pltpu.VMEM((SEG,), jnp.float32),
        pltpu.VMEM_SHARED((NPAD,), jnp.float32),
        pltpu.VMEM_SHARED((NPAD,), jnp.float32),
        pltpu.SemaphoreType.DMA,
    ],
)(_sc_degrees_body)


_sc_scatter = functools.partial(
    pl.kernel,
    out_type=jax.ShapeDtypeStruct((NC, 2, NPAD), jnp.float32),
    mesh=plsc.VectorSubcoreMesh(
        core_axis_name="c", subcore_axis_name="s", num_cores=NC, num_subcores=NS
    ),
    compiler_params=pltpu.CompilerParams(use_tc_tiling_on_sc=False, needs_layout_passes=False),
    scratch_types=[
        pltpu.VMEM((RPW + 1, LANES), jnp.int32),
        pltpu.VMEM((RPW + 1, LANES), jnp.int32),
        pltpu.VMEM((RPW + 1, LANES), jnp.float32),
        pltpu.VMEM((RPW + 1, LANES), jnp.float32),
        pltpu.VMEM((NPAD,), jnp.float32),
        pltpu.VMEM((NPAD,), jnp.float32),
        pltpu.VMEM((SEG,), jnp.float32),
        pltpu.VMEM((SEG,), jnp.float32),
        pltpu.VMEM((SEG,), jnp.float32),
        pltpu.VMEM((SEG,), jnp.float32),
        pltpu.VMEM((SEG,), jnp.float32),
        pltpu.VMEM((SEG,), jnp.float32),
        pltpu.VMEM((SEG,), jnp.float32),
        pltpu.VMEM_SHARED((NPAD,), jnp.float32),
        pltpu.VMEM_SHARED((NPAD,), jnp.float32),
        pltpu.VMEM_SHARED((NPAD,), jnp.float32),
        pltpu.VMEM_SHARED((NPAD,), jnp.float32),
        pltpu.SemaphoreType.DMA,
    ],
)(_sc_scatter_body)


def _tc_u_body(x_ref, we_ref, fw_ref, wg_ref, cw_ref, u2_ref):
  v = jnp.concatenate(
      [
          jnp.dot(we_ref[...], fw_ref[...], preferred_element_type=jnp.float32),
          jnp.dot(wg_ref[...], cw_ref[...], preferred_element_type=jnp.float32),
      ],
      axis=1,
  )
  u_t = lax.dot_general(v, x_ref[...], (((0,), (1,)), ((), ())),
                        preferred_element_type=jnp.float32)
  u2_ref[...] = jnp.concatenate(
      [u_t, jnp.zeros((2, NPAD - N), jnp.float32)], axis=1
  )


def _tc_final_body(accp_ref, degs_ref, be_ref, fw_ref, fb_ref, bg_ref, cw_ref,
                   cb_ref, y_ref, s_ref):
  deg_i = degs_ref[0, 1, :N] + degs_ref[1, 1, :N]
  nd = jnp.where(deg_i > 0, lax.rsqrt(jnp.maximum(deg_i, 1.0)), 0.0)
  acc_a = accp_ref[0, 0, :N] + accp_ref[1, 0, :N]
  acc_b = accp_ref[0, 1, :N] + accp_ref[1, 1, :N]
  c_est = jnp.dot(be_ref[...][None, :], fw_ref[...],
                  preferred_element_type=jnp.float32)[0, 0] + fb_ref[0]
  c_gnn = jnp.dot(bg_ref[...][None, :], cw_ref[...],
                  preferred_element_type=jnp.float32)[0, 0] + cb_ref[0]
  s_ref[...] = (acc_a * nd + c_est)[:, None]
  y_ref[...] = (acc_b * nd + c_gnn)[:, None]


_tc_u = pl.pallas_call(
    _tc_u_body,
    out_shape=jax.ShapeDtypeStruct((2, NPAD), jnp.float32),
)

_tc_final = pl.pallas_call(
    _tc_final_body,
    out_shape=[
        jax.ShapeDtypeStruct((N, 1), jnp.float32),
        jax.ShapeDtypeStruct((N, 1), jnp.float32),
    ],
)


def kernel(x, edge_index, W_est, b_est, fc_w, fc_b, W_gnn, b_gnn, cls_w, cls_b):
  e2 = edge_index.reshape(2, ROWS, LANES)
  u2 = _tc_u(x, W_est, fc_w, W_gnn, cls_w)
  degs = _sc_degrees(e2)
  accp = _sc_scatter(e2, u2, degs)
  y, s = _tc_final(accp, degs, b_est, fc_w, fc_b, b_gnn, cls_w, cls_b)
  return (y, s)
